# baseline (speedup 1.0000x reference)
import functools

import jax
import jax.numpy as jnp
from jax.experimental import pallas as pl


def _assign_body(x_ref, c_ref, o_ref):
    xt = x_ref[...]
    ct = c_ref[...]
    xc = jax.lax.dot_general(
        xt, ct, (((1,), (1,)), ((), ())),
        preferred_element_type=jnp.float32)
    x2 = jnp.sum(xt * xt, axis=1, keepdims=True)
    c2 = jnp.sum(ct * ct, axis=1)[None, :]
    d = x2 - 2.0 * xc + c2
    o_ref[0, 0, :] = jnp.argmin(d, axis=1).astype(jnp.int32)


@functools.partial(jax.jit, static_argnames=("interpret",))
def _assign(x, centers, interpret=False):
    B, S, D = x.shape
    K = centers.shape[0]
    N = B * S
    M = 1024
    G = N // M
    xf = x.reshape(N, D)
    out = pl.pallas_call(
        _assign_body,
        grid=(G,),
        in_specs=[
            pl.BlockSpec((M, D), lambda i: (i, 0)),
            pl.BlockSpec((K, D), lambda i: (0, 0)),
        ],
        out_specs=pl.BlockSpec((1, 1, M), lambda i: (i, 0, 0)),
        out_shape=jax.ShapeDtypeStruct((G, 1, M), jnp.int32),
        interpret=interpret,
    )(xf, centers)
    return out.reshape(B, S).astype(jnp.int64)


def kernel(x, centers):
    return _assign(x, centers)
